# baseline (speedup 1.0000x reference)
import functools

import jax
import jax.numpy as jnp
from jax.experimental import pallas as pl
from jax.experimental.pallas import tpu as pltpu
from jax.experimental.pallas import tpu_sc as plsc
from jax.experimental.shard_map import shard_map as _shard_map

NX = 16
IN_CH = 32
OUT_CH = 64
N_PTS = 4096
DX = 2.0 / NX
RADIUS = DX * 2.5
EPS = 1e-12

GRID_CHUNK = 256
PT_TILE = 256
N_TILES = N_PTS // PT_TILE


def _taps(u):
    w0 = jnp.maximum(0.0, -u)
    w1 = 1.0 - jnp.abs(u)
    w2 = jnp.maximum(0.0, u)
    return w0, w1, w2


_ATAN_C = (0.99999611, -0.33317368, 0.19807815, -0.13233339,
           0.07962363, -0.03360418, 0.00681178)


def _atan_poly(t):
    t2 = t * t
    acc = jnp.float32(_ATAN_C[-1])
    for c in _ATAN_C[-2::-1]:
        acc = jnp.float32(c) + t2 * acc
    return t * acc


def _ball_to_cube(x, y, z, rxy2, sq, zz):
    norm = jnp.sqrt(sq + EPS)
    polar = (1.25 * zz) > rxy2
    s_pol = jnp.sqrt(3.0 * norm / (norm + jnp.abs(z) + EPS))
    cx_p = x * s_pol
    cy_p = y * s_pol
    cz_p = jnp.sign(z) * norm
    rxy = jnp.sqrt(rxy2 + EPS)
    s_eq = norm / rxy
    cx_e = x * s_eq
    cy_e = y * s_eq
    cz_e = 1.5 * z
    cx = jnp.where(polar, cx_p, cx_e)
    cy = jnp.where(polar, cy_p, cy_e)
    cz = jnp.where(polar, cz_p, cz_e)
    degen = sq < 1e-10
    cx = jnp.where(degen, 0.0, cx)
    cy = jnp.where(degen, 0.0, cy)
    cz = jnp.where(degen, 0.0, cz)
    rsq = cx * cx + cy * cy
    rn = jnp.sqrt(rsq + EPS)
    cond = cx * cx >= cy * cy
    safe_cx = jnp.where(jnp.abs(cx) > 1e-6, cx, 1.0)
    safe_cy = jnp.where(jnp.abs(cy) > 1e-6, cy, 1.0)
    pi4 = 4.0 / jnp.pi
    sgn_cx = jnp.sign(cx)
    sgn_cy = jnp.sign(cy)
    u1 = sgn_cx * rn
    v1 = sgn_cx * rn * pi4 * _atan_poly(cy / safe_cx)
    v2 = sgn_cy * rn
    u2 = sgn_cy * rn * pi4 * _atan_poly(cx / safe_cy)
    u = jnp.where(cond, u1, u2)
    v = jnp.where(cond, v1, v2)
    small = rsq < 1e-10
    u = jnp.where(small, 0.0, u)
    v = jnp.where(small, 0.0, v)
    return (jnp.clip(u, -1.0, 1.0), jnp.clip(v, -1.0, 1.0),
            jnp.clip(cz, -1.0, 1.0))


SORT_BLK = 512
DATA_COLS = 128
SCAT_WIN = 128


def _bin_kernel(pos_x_ref, rank_ref, starts_ref):
    x = pos_x_ref[0:1, :]
    cf = jnp.clip(jnp.floor((x + 1.0) * (NX / 2.0)), 0.0, NX - 1.0)
    c_col = cf.reshape(N_PTS, 1).astype(jnp.int32)
    lane16 = jax.lax.broadcasted_iota(jnp.int32, (N_PTS, NX), 1)
    onehot = jnp.where(c_col == lane16, 1.0, 0.0)

    r_i = jax.lax.broadcasted_iota(jnp.int32, (SORT_BLK, SORT_BLK), 0)
    c_i = jax.lax.broadcasted_iota(jnp.int32, (SORT_BLK, SORT_BLK), 1)
    tri = jnp.where(r_i >= c_i, 1.0, 0.0).astype(jnp.bfloat16)

    carry = jnp.zeros((1, NX), jnp.float32)
    cums_blocks = []
    for b in range(N_PTS // SORT_BLK):
        blk = onehot[b * SORT_BLK:(b + 1) * SORT_BLK, :]
        within = jnp.dot(tri, blk.astype(jnp.bfloat16),
                         preferred_element_type=jnp.float32)
        cums_blocks.append(within + carry)
        carry = carry + within[SORT_BLK - 1:SORT_BLK, :]
    cums = jnp.concatenate(cums_blocks, axis=0)

    tot_col = carry.reshape(NX, 1)
    t_sub = jax.lax.broadcasted_iota(jnp.int32, (NX, 32), 0)
    s_lane = jax.lax.broadcasted_iota(jnp.int32, (NX, 32), 1)
    starts32 = jnp.sum(jnp.where(t_sub < s_lane, tot_col, 0.0), axis=0,
                       keepdims=True)
    starts_ref[...] = starts32.astype(jnp.int32)

    start_g = jnp.sum(onehot * starts32[:, 0:NX], axis=1, keepdims=True)
    dup_g = jnp.sum(onehot * cums, axis=1, keepdims=True) - 1.0
    rank_col = start_g + dup_g
    rank_ref[...] = rank_col.reshape(1, N_PTS).astype(jnp.int32)


@jax.jit
def _bin_ranks(pos_x):
    return pl.pallas_call(
        _bin_kernel,
        in_specs=[pl.BlockSpec((1, N_PTS), lambda: (0, 0))],
        out_specs=[pl.BlockSpec((1, N_PTS), lambda: (0, 0)),
                   pl.BlockSpec((1, 32), lambda: (0, 0))],
        out_shape=[jax.ShapeDtypeStruct((1, N_PTS), jnp.int32),
                   jax.ShapeDtypeStruct((1, 32), jnp.int32)],
    )(pos_x)


@jax.jit
def _sc_scatter_rows(data, rank):
    mesh = plsc.VectorSubcoreMesh(core_axis_name="core",
                                  subcore_axis_name="subcore")

    @functools.partial(
        pl.kernel,
        out_type=jax.ShapeDtypeStruct((N_PTS, DATA_COLS), jnp.float32),
        mesh=mesh)
    def scatter_kernel(x_hbm, i_hbm, o_hbm):
        def body(x_vmem, i_vmem):
            pltpu.sync_copy(x_vmem, o_hbm.at[i_vmem.at[0]])

        pltpu.emit_pipeline(
            body,
            grid=(N_PTS // SCAT_WIN,),
            in_specs=[pl.BlockSpec((SCAT_WIN, DATA_COLS),
                                   index_map=lambda i: (i, 0)),
                      pl.BlockSpec((1, SCAT_WIN),
                                   index_map=lambda i: (0, i))],
            out_specs=[],
            core_axis_name=("core", "subcore"),
            dimension_semantics=(pltpu.PARALLEL,),
        )(x_hbm, i_hbm)

    return scatter_kernel(data, rank)


def _cconv_kernel(starts_ref, gp_ref, pos_ref, feat_ref, wk_ref, out_ref,
                  agg_ref, nbr_ref):
    agg_ref[...] = jnp.zeros_like(agg_ref)
    nbr_ref[...] = jnp.zeros_like(nbr_ref)

    gx = gp_ref[0, :].reshape(GRID_CHUNK, 1)
    gy = gp_ref[1, :].reshape(GRID_CHUNK, 1)
    gz = gp_ref[2, :].reshape(GRID_CHUNK, 1)
    inv_r = 1.0 / RADIUS

    ix = ((gp_ref[0, 0] + 1.0) * (NX / 2.0)).astype(jnp.int32)
    lo = starts_ref[jnp.maximum(ix - 2, 0)]
    hi = starts_ref[jnp.minimum(ix + 3, NX)]
    t0 = lo // PT_TILE
    t1 = (hi + PT_TILE - 1) // PT_TILE

    def tile_body(t, _):
        sl = pl.ds(t * PT_TILE, PT_TILE)
        px = pos_ref[0, sl].reshape(1, PT_TILE)
        py = pos_ref[1, sl].reshape(1, PT_TILE)
        pz = pos_ref[2, sl].reshape(1, PT_TILE)
        feat_t = feat_ref[sl, :]

        x = (px - gx) * inv_r
        y = (py - gy) * inv_r
        z = (pz - gz) * inv_r
        rxy2 = x * x + y * y
        zz = z * z
        sq = rxy2 + zz
        maskf = jnp.where(sq <= 1.0, 1.0, 0.0)
        nbr_ref[...] += maskf

        u, v, w = _ball_to_cube(x, y, z, rxy2, sq, zz)
        wx = _taps(u)
        wy = _taps(v)
        wz = _taps(w)
        wzm = [(wz[k] * maskf).astype(jnp.bfloat16) for k in range(3)]
        for i in range(3):
            for j in range(3):
                wxy = (wx[i] * wy[j]).astype(jnp.bfloat16)
                for k in range(3):
                    kk = (i * 3 + j) * 3 + k
                    w27 = wxy * wzm[k]
                    agg_ref[:, kk * IN_CH:(kk + 1) * IN_CH] += jnp.dot(
                        w27, feat_t, preferred_element_type=jnp.float32)
        return _

    jax.lax.fori_loop(t0, t1, tile_body, 0)

    acc = jnp.dot(agg_ref[...].astype(jnp.bfloat16), wk_ref[...],
                  preferred_element_type=jnp.float32)
    nbr = jnp.maximum(jnp.sum(nbr_ref[...], axis=1), 1.0)
    out_ref[...] = acc / nbr.reshape(GRID_CHUNK, 1)


def _cconv(starts, feat, pos_t, wk_flat, grid_pos_t, n_chunks):
    grid_spec = pltpu.PrefetchScalarGridSpec(
        num_scalar_prefetch=1,
        grid=(n_chunks,),
        in_specs=[
            pl.BlockSpec((3, GRID_CHUNK), lambda i, s: (0, i)),
            pl.BlockSpec((3, N_PTS), lambda i, s: (0, 0)),
            pl.BlockSpec((N_PTS, IN_CH), lambda i, s: (0, 0)),
            pl.BlockSpec((27 * IN_CH, OUT_CH), lambda i, s: (0, 0)),
        ],
        out_specs=pl.BlockSpec((GRID_CHUNK, OUT_CH), lambda i, s: (i, 0)),
        scratch_shapes=[
            pltpu.VMEM((GRID_CHUNK, 27 * IN_CH), jnp.float32),
            pltpu.VMEM((GRID_CHUNK, PT_TILE), jnp.float32),
        ],
    )
    out = pl.pallas_call(
        _cconv_kernel,
        grid_spec=grid_spec,
        out_shape=jax.ShapeDtypeStruct((n_chunks * GRID_CHUNK, OUT_CH),
                                       jnp.float32),
    )(starts, grid_pos_t, pos_t, feat, wk_flat)
    return out


def _full_pipeline(p, feat_in, wk, grid_pos_t, n_chunks):
    rank, starts = _bin_ranks(p[:, 0].reshape(1, N_PTS))
    data = jnp.concatenate(
        [p, jnp.zeros((N_PTS, 16 - 3), jnp.float32), feat_in,
         jnp.zeros((N_PTS, DATA_COLS - 48), jnp.float32)], axis=1)
    sorted_data = _sc_scatter_rows(data, rank)
    pos_t = sorted_data[:, 0:3].T
    feat = sorted_data[:, 16:48].astype(jnp.bfloat16)
    wk_flat = wk.reshape(27 * IN_CH, OUT_CH).astype(jnp.bfloat16)
    return _cconv(starts.reshape(32), feat, pos_t, wk_flat, grid_pos_t,
                  n_chunks)


def kernel(input, pos, kernel, grid_pos):
    p = pos[0]
    grid_pos_t = grid_pos.T

    n_total = NX * NX * NX // GRID_CHUNK
    n_dev = 1
    for d in (2, 4):
        if len(jax.devices()) % d == 0 and n_total % d == 0:
            n_dev = d
    if n_dev > 1:
        mesh = jax.sharding.Mesh(jax.devices()[:n_dev], ("d",))
        pspec = jax.sharding.PartitionSpec
        out = _shard_map(
            functools.partial(_full_pipeline, n_chunks=n_total // n_dev),
            mesh=mesh,
            in_specs=(pspec(), pspec(), pspec(), pspec(None, "d")),
            out_specs=pspec("d"),
            check_rep=False,
        )(p, input[0], kernel, grid_pos_t)
    else:
        out = _full_pipeline(p, input[0], kernel, grid_pos_t, n_total)
    grid_feat = out.reshape(1, NX, NX, NX, OUT_CH)
    return jnp.transpose(grid_feat, (0, 4, 1, 2, 3))


# speedup vs baseline: 2.2411x; 2.2411x over previous
import functools

import jax
import jax.numpy as jnp
from jax.experimental import pallas as pl
from jax.experimental.pallas import tpu as pltpu
from jax.experimental.pallas import tpu_sc as plsc

NX = 16
IN_CH = 32
OUT_CH = 64
N_PTS = 4096
DX = 2.0 / NX
RADIUS = DX * 2.5
EPS = 1e-12

GRID_CHUNK = 256
PT_TILE = 256
N_TILES = N_PTS // PT_TILE


def _taps(u):
    w0 = jnp.maximum(0.0, -u)
    w1 = 1.0 - jnp.abs(u)
    w2 = jnp.maximum(0.0, u)
    return w0, w1, w2


_ATAN_C = (0.99999611, -0.33317368, 0.19807815, -0.13233339,
           0.07962363, -0.03360418, 0.00681178)


def _atan_poly(t):
    t2 = t * t
    acc = jnp.float32(_ATAN_C[-1])
    for c in _ATAN_C[-2::-1]:
        acc = jnp.float32(c) + t2 * acc
    return t * acc


def _ball_to_cube(x, y, z, rxy2, sq, zz):
    norm = jnp.sqrt(sq + EPS)
    polar = (1.25 * zz) > rxy2
    s_pol = jnp.sqrt(3.0 * norm / (norm + jnp.abs(z) + EPS))
    cx_p = x * s_pol
    cy_p = y * s_pol
    cz_p = jnp.sign(z) * norm
    rxy = jnp.sqrt(rxy2 + EPS)
    s_eq = norm / rxy
    cx_e = x * s_eq
    cy_e = y * s_eq
    cz_e = 1.5 * z
    cx = jnp.where(polar, cx_p, cx_e)
    cy = jnp.where(polar, cy_p, cy_e)
    cz = jnp.where(polar, cz_p, cz_e)
    degen = sq < 1e-10
    cx = jnp.where(degen, 0.0, cx)
    cy = jnp.where(degen, 0.0, cy)
    cz = jnp.where(degen, 0.0, cz)
    rsq = cx * cx + cy * cy
    rn = jnp.sqrt(rsq + EPS)
    cond = cx * cx >= cy * cy
    safe_cx = jnp.where(jnp.abs(cx) > 1e-6, cx, 1.0)
    safe_cy = jnp.where(jnp.abs(cy) > 1e-6, cy, 1.0)
    pi4 = 4.0 / jnp.pi
    sgn_cx = jnp.sign(cx)
    sgn_cy = jnp.sign(cy)
    u1 = sgn_cx * rn
    v1 = sgn_cx * rn * pi4 * _atan_poly(cy / safe_cx)
    v2 = sgn_cy * rn
    u2 = sgn_cy * rn * pi4 * _atan_poly(cx / safe_cy)
    u = jnp.where(cond, u1, u2)
    v = jnp.where(cond, v1, v2)
    small = rsq < 1e-10
    u = jnp.where(small, 0.0, u)
    v = jnp.where(small, 0.0, v)
    return (jnp.clip(u, -1.0, 1.0), jnp.clip(v, -1.0, 1.0),
            jnp.clip(cz, -1.0, 1.0))


SORT_BLK = 512
DATA_COLS = 128
SCAT_WIN = 128


def _bin_kernel(pos_x_ref, rank_ref, starts_ref):
    x = pos_x_ref[0:1, :]
    cf = jnp.clip(jnp.floor((x + 1.0) * (NX / 2.0)), 0.0, NX - 1.0)
    c_col = cf.reshape(N_PTS, 1).astype(jnp.int32)
    lane16 = jax.lax.broadcasted_iota(jnp.int32, (N_PTS, NX), 1)
    onehot = jnp.where(c_col == lane16, 1.0, 0.0)

    r_i = jax.lax.broadcasted_iota(jnp.int32, (SORT_BLK, SORT_BLK), 0)
    c_i = jax.lax.broadcasted_iota(jnp.int32, (SORT_BLK, SORT_BLK), 1)
    tri = jnp.where(r_i >= c_i, 1.0, 0.0).astype(jnp.bfloat16)

    carry = jnp.zeros((1, NX), jnp.float32)
    cums_blocks = []
    for b in range(N_PTS // SORT_BLK):
        blk = onehot[b * SORT_BLK:(b + 1) * SORT_BLK, :]
        within = jnp.dot(tri, blk.astype(jnp.bfloat16),
                         preferred_element_type=jnp.float32)
        cums_blocks.append(within + carry)
        carry = carry + within[SORT_BLK - 1:SORT_BLK, :]
    cums = jnp.concatenate(cums_blocks, axis=0)

    tot_col = carry.reshape(NX, 1)
    t_sub = jax.lax.broadcasted_iota(jnp.int32, (NX, 32), 0)
    s_lane = jax.lax.broadcasted_iota(jnp.int32, (NX, 32), 1)
    starts32 = jnp.sum(jnp.where(t_sub < s_lane, tot_col, 0.0), axis=0,
                       keepdims=True)
    starts_ref[...] = starts32.astype(jnp.int32)

    start_g = jnp.sum(onehot * starts32[:, 0:NX], axis=1, keepdims=True)
    dup_g = jnp.sum(onehot * cums, axis=1, keepdims=True) - 1.0
    rank_col = start_g + dup_g
    rank_ref[...] = rank_col.reshape(1, N_PTS).astype(jnp.int32)


@jax.jit
def _bin_ranks(pos_x):
    return pl.pallas_call(
        _bin_kernel,
        in_specs=[pl.BlockSpec((1, N_PTS), lambda: (0, 0))],
        out_specs=[pl.BlockSpec((1, N_PTS), lambda: (0, 0)),
                   pl.BlockSpec((1, 32), lambda: (0, 0))],
        out_shape=[jax.ShapeDtypeStruct((1, N_PTS), jnp.int32),
                   jax.ShapeDtypeStruct((1, 32), jnp.int32)],
    )(pos_x)


@jax.jit
def _sc_scatter_rows(data, rank):
    mesh = plsc.VectorSubcoreMesh(core_axis_name="core",
                                  subcore_axis_name="subcore")

    @functools.partial(
        pl.kernel,
        out_type=jax.ShapeDtypeStruct((N_PTS, DATA_COLS), jnp.float32),
        mesh=mesh)
    def scatter_kernel(x_hbm, i_hbm, o_hbm):
        def body(x_vmem, i_vmem):
            pltpu.sync_copy(x_vmem, o_hbm.at[i_vmem.at[0]])

        pltpu.emit_pipeline(
            body,
            grid=(N_PTS // SCAT_WIN,),
            in_specs=[pl.BlockSpec((SCAT_WIN, DATA_COLS),
                                   index_map=lambda i: (i, 0)),
                      pl.BlockSpec((1, SCAT_WIN),
                                   index_map=lambda i: (0, i))],
            out_specs=[],
            core_axis_name=("core", "subcore"),
            dimension_semantics=(pltpu.PARALLEL,),
        )(x_hbm, i_hbm)

    return scatter_kernel(data, rank)


def _cconv_kernel(starts_ref, gp_ref, pos_ref, feat_ref, wk_ref, out_ref,
                  agg_ref, nbr_ref):
    agg_ref[...] = jnp.zeros_like(agg_ref)
    nbr_ref[...] = jnp.zeros_like(nbr_ref)

    gx = gp_ref[0, :].reshape(GRID_CHUNK, 1)
    gy = gp_ref[1, :].reshape(GRID_CHUNK, 1)
    gz = gp_ref[2, :].reshape(GRID_CHUNK, 1)
    inv_r = 1.0 / RADIUS

    ix = ((gp_ref[0, 0] + 1.0) * (NX / 2.0)).astype(jnp.int32)
    lo = starts_ref[jnp.maximum(ix - 2, 0)]
    hi = starts_ref[jnp.minimum(ix + 3, NX)]
    t0 = lo // PT_TILE
    t1 = (hi + PT_TILE - 1) // PT_TILE

    def tile_body(t, _):
        sl = pl.ds(t * PT_TILE, PT_TILE)
        px = pos_ref[0, sl].reshape(1, PT_TILE)
        py = pos_ref[1, sl].reshape(1, PT_TILE)
        pz = pos_ref[2, sl].reshape(1, PT_TILE)
        feat_t = feat_ref[sl, :]

        x = (px - gx) * inv_r
        y = (py - gy) * inv_r
        z = (pz - gz) * inv_r
        rxy2 = x * x + y * y
        zz = z * z
        sq = rxy2 + zz
        maskf = jnp.where(sq <= 1.0, 1.0, 0.0)
        nbr_ref[...] += maskf

        u, v, w = _ball_to_cube(x, y, z, rxy2, sq, zz)
        wx = _taps(u)
        wy = _taps(v)
        wz = _taps(w)
        wzm = [(wz[k] * maskf).astype(jnp.bfloat16) for k in range(3)]
        for i in range(3):
            for j in range(3):
                wxy = (wx[i] * wy[j]).astype(jnp.bfloat16)
                for k in range(3):
                    kk = (i * 3 + j) * 3 + k
                    w27 = wxy * wzm[k]
                    agg_ref[:, kk * IN_CH:(kk + 1) * IN_CH] += jnp.dot(
                        w27, feat_t, preferred_element_type=jnp.float32)
        return _

    jax.lax.fori_loop(t0, t1, tile_body, 0)

    acc = jnp.dot(agg_ref[...].astype(jnp.bfloat16), wk_ref[...],
                  preferred_element_type=jnp.float32)
    nbr = jnp.maximum(jnp.sum(nbr_ref[...], axis=1), 1.0)
    out_ref[...] = acc / nbr.reshape(GRID_CHUNK, 1)


def _cconv(starts, feat, pos_t, wk_flat, grid_pos_t, n_chunks):
    grid_spec = pltpu.PrefetchScalarGridSpec(
        num_scalar_prefetch=1,
        grid=(n_chunks,),
        in_specs=[
            pl.BlockSpec((3, GRID_CHUNK), lambda i, s: (0, i)),
            pl.BlockSpec((3, N_PTS), lambda i, s: (0, 0)),
            pl.BlockSpec((N_PTS, IN_CH), lambda i, s: (0, 0)),
            pl.BlockSpec((27 * IN_CH, OUT_CH), lambda i, s: (0, 0)),
        ],
        out_specs=pl.BlockSpec((GRID_CHUNK, OUT_CH), lambda i, s: (i, 0)),
        scratch_shapes=[
            pltpu.VMEM((GRID_CHUNK, 27 * IN_CH), jnp.float32),
            pltpu.VMEM((GRID_CHUNK, PT_TILE), jnp.float32),
        ],
    )
    out = pl.pallas_call(
        _cconv_kernel,
        grid_spec=grid_spec,
        out_shape=jax.ShapeDtypeStruct((n_chunks * GRID_CHUNK, OUT_CH),
                                       jnp.float32),
    )(starts, grid_pos_t, pos_t, feat, wk_flat)
    return out


def _full_pipeline(p, feat_in, wk, grid_pos_t, n_chunks):
    rank, starts = _bin_ranks(p[:, 0].reshape(1, N_PTS))
    data = jnp.concatenate(
        [p, jnp.zeros((N_PTS, 16 - 3), jnp.float32), feat_in,
         jnp.zeros((N_PTS, DATA_COLS - 48), jnp.float32)], axis=1)
    sorted_data = _sc_scatter_rows(data, rank)
    pos_t = sorted_data[:, 0:3].T
    feat = sorted_data[:, 16:48].astype(jnp.bfloat16)
    wk_flat = wk.reshape(27 * IN_CH, OUT_CH).astype(jnp.bfloat16)
    return _cconv(starts.reshape(32), feat, pos_t, wk_flat, grid_pos_t,
                  n_chunks)


def kernel(input, pos, kernel, grid_pos):
    p = pos[0]
    grid_pos_t = grid_pos.T

    n_total = NX * NX * NX // GRID_CHUNK
    out = _full_pipeline(p, input[0], kernel, grid_pos_t, n_total)
    grid_feat = out.reshape(1, NX, NX, NX, OUT_CH)
    return jnp.transpose(grid_feat, (0, 4, 1, 2, 3))


# speedup vs baseline: 2.3901x; 1.0665x over previous
import functools

import jax
import jax.numpy as jnp
from jax.experimental import pallas as pl
from jax.experimental.pallas import tpu as pltpu
from jax.experimental.pallas import tpu_sc as plsc

NX = 16
IN_CH = 32
OUT_CH = 64
N_PTS = 4096
DX = 2.0 / NX
RADIUS = DX * 2.5
EPS = 1e-12

GRID_CHUNK = 256
PT_TILE = 256
N_TILES = N_PTS // PT_TILE


def _taps(u):
    w0 = jnp.maximum(0.0, -u)
    w1 = 1.0 - jnp.abs(u)
    w2 = jnp.maximum(0.0, u)
    return w0, w1, w2


_ATAN_C = (0.99999611, -0.33317368, 0.19807815, -0.13233339,
           0.07962363, -0.03360418, 0.00681178)


def _atan_poly(t):
    t2 = t * t
    acc = jnp.float32(_ATAN_C[-1])
    for c in _ATAN_C[-2::-1]:
        acc = jnp.float32(c) + t2 * acc
    return t * acc


def _ball_to_cube(x, y, z, rxy2, sq, zz):
    norm = jnp.sqrt(sq + EPS)
    polar = (1.25 * zz) > rxy2
    s_pol = jnp.sqrt(3.0 * norm / (norm + jnp.abs(z) + EPS))
    cx_p = x * s_pol
    cy_p = y * s_pol
    cz_p = jnp.sign(z) * norm
    rxy = jnp.sqrt(rxy2 + EPS)
    s_eq = norm / rxy
    cx_e = x * s_eq
    cy_e = y * s_eq
    cz_e = 1.5 * z
    cx = jnp.where(polar, cx_p, cx_e)
    cy = jnp.where(polar, cy_p, cy_e)
    cz = jnp.where(polar, cz_p, cz_e)
    degen = sq < 1e-10
    cx = jnp.where(degen, 0.0, cx)
    cy = jnp.where(degen, 0.0, cy)
    cz = jnp.where(degen, 0.0, cz)
    rsq = cx * cx + cy * cy
    rn = jnp.sqrt(rsq + EPS)
    cond = cx * cx >= cy * cy
    safe_cx = jnp.where(jnp.abs(cx) > 1e-6, cx, 1.0)
    safe_cy = jnp.where(jnp.abs(cy) > 1e-6, cy, 1.0)
    pi4 = 4.0 / jnp.pi
    sgn_cx = jnp.sign(cx)
    sgn_cy = jnp.sign(cy)
    u1 = sgn_cx * rn
    v1 = sgn_cx * rn * pi4 * _atan_poly(cy / safe_cx)
    v2 = sgn_cy * rn
    u2 = sgn_cy * rn * pi4 * _atan_poly(cx / safe_cy)
    u = jnp.where(cond, u1, u2)
    v = jnp.where(cond, v1, v2)
    small = rsq < 1e-10
    u = jnp.where(small, 0.0, u)
    v = jnp.where(small, 0.0, v)
    return u, v, cz


SORT_BLK = 512
DATA_COLS = 128
SCAT_WIN = 128


def _bin_kernel(pos_x_ref, rank_ref, starts_ref):
    x = pos_x_ref[0:1, :]
    cf = jnp.clip(jnp.floor((x + 1.0) * (NX / 2.0)), 0.0, NX - 1.0)
    c_col = cf.reshape(N_PTS, 1).astype(jnp.int32)
    lane16 = jax.lax.broadcasted_iota(jnp.int32, (N_PTS, NX), 1)
    onehot = jnp.where(c_col == lane16, 1.0, 0.0)

    r_i = jax.lax.broadcasted_iota(jnp.int32, (SORT_BLK, SORT_BLK), 0)
    c_i = jax.lax.broadcasted_iota(jnp.int32, (SORT_BLK, SORT_BLK), 1)
    tri = jnp.where(r_i >= c_i, 1.0, 0.0).astype(jnp.bfloat16)

    carry = jnp.zeros((1, NX), jnp.float32)
    cums_blocks = []
    for b in range(N_PTS // SORT_BLK):
        blk = onehot[b * SORT_BLK:(b + 1) * SORT_BLK, :]
        within = jnp.dot(tri, blk.astype(jnp.bfloat16),
                         preferred_element_type=jnp.float32)
        cums_blocks.append(within + carry)
        carry = carry + within[SORT_BLK - 1:SORT_BLK, :]
    cums = jnp.concatenate(cums_blocks, axis=0)

    tot_col = carry.reshape(NX, 1)
    t_sub = jax.lax.broadcasted_iota(jnp.int32, (NX, 32), 0)
    s_lane = jax.lax.broadcasted_iota(jnp.int32, (NX, 32), 1)
    starts32 = jnp.sum(jnp.where(t_sub < s_lane, tot_col, 0.0), axis=0,
                       keepdims=True)
    starts_ref[...] = starts32.astype(jnp.int32)

    start_g = jnp.sum(onehot * starts32[:, 0:NX], axis=1, keepdims=True)
    dup_g = jnp.sum(onehot * cums, axis=1, keepdims=True) - 1.0
    rank_col = start_g + dup_g
    rank_ref[...] = rank_col.reshape(1, N_PTS).astype(jnp.int32)


@jax.jit
def _bin_ranks(pos_x):
    return pl.pallas_call(
        _bin_kernel,
        in_specs=[pl.BlockSpec((1, N_PTS), lambda: (0, 0))],
        out_specs=[pl.BlockSpec((1, N_PTS), lambda: (0, 0)),
                   pl.BlockSpec((1, 32), lambda: (0, 0))],
        out_shape=[jax.ShapeDtypeStruct((1, N_PTS), jnp.int32),
                   jax.ShapeDtypeStruct((1, 32), jnp.int32)],
    )(pos_x)


@jax.jit
def _sc_scatter_rows(data, rank):
    mesh = plsc.VectorSubcoreMesh(core_axis_name="core",
                                  subcore_axis_name="subcore")

    @functools.partial(
        pl.kernel,
        out_type=jax.ShapeDtypeStruct((N_PTS, DATA_COLS), jnp.float32),
        mesh=mesh)
    def scatter_kernel(x_hbm, i_hbm, o_hbm):
        def body(x_vmem, i_vmem):
            pltpu.sync_copy(x_vmem, o_hbm.at[i_vmem.at[0]])

        pltpu.emit_pipeline(
            body,
            grid=(N_PTS // SCAT_WIN,),
            in_specs=[pl.BlockSpec((SCAT_WIN, DATA_COLS),
                                   index_map=lambda i: (i, 0)),
                      pl.BlockSpec((1, SCAT_WIN),
                                   index_map=lambda i: (0, i))],
            out_specs=[],
            core_axis_name=("core", "subcore"),
            dimension_semantics=(pltpu.PARALLEL,),
        )(x_hbm, i_hbm)

    return scatter_kernel(data, rank)


def _cconv_kernel(starts_ref, gp_ref, pos_ref, feat_ref, wk_ref, out_ref,
                  agg_ref, nbr_ref):
    agg_ref[...] = jnp.zeros_like(agg_ref)
    nbr_ref[...] = jnp.zeros_like(nbr_ref)

    gx = gp_ref[0, :].reshape(GRID_CHUNK, 1)
    gy = gp_ref[1, :].reshape(GRID_CHUNK, 1)
    gz = gp_ref[2, :].reshape(GRID_CHUNK, 1)
    inv_r = 1.0 / RADIUS

    ix = ((gp_ref[0, 0] + 1.0) * (NX / 2.0)).astype(jnp.int32)
    lo = starts_ref[jnp.maximum(ix - 2, 0)]
    hi = starts_ref[jnp.minimum(ix + 3, NX)]
    t0 = lo // PT_TILE
    t1 = (hi + PT_TILE - 1) // PT_TILE

    def tile_body(t, _):
        sl = pl.ds(t * PT_TILE, PT_TILE)
        px = pos_ref[0, sl].reshape(1, PT_TILE)
        py = pos_ref[1, sl].reshape(1, PT_TILE)
        pz = pos_ref[2, sl].reshape(1, PT_TILE)
        feat_t = feat_ref[sl, :]

        x = (px - gx) * inv_r
        y = (py - gy) * inv_r
        z = (pz - gz) * inv_r
        rxy2 = x * x + y * y
        zz = z * z
        sq = rxy2 + zz
        maskf = jnp.where(sq <= 1.0, 1.0, 0.0)
        nbr_ref[...] += maskf

        u, v, w = _ball_to_cube(x, y, z, rxy2, sq, zz)
        wx = _taps(u)
        wy = _taps(v)
        wz = _taps(w)
        wzm = [(wz[k] * maskf).astype(jnp.bfloat16) for k in range(3)]
        ws = []
        for i in range(3):
            for j in range(3):
                wxy = (wx[i] * wy[j]).astype(jnp.bfloat16)
                for k in range(3):
                    ws.append(wxy * wzm[k])
        w_all = jnp.concatenate(ws, axis=0)
        agg_ref[...] += jnp.dot(w_all, feat_t,
                                preferred_element_type=jnp.float32)
        return _

    jax.lax.fori_loop(t0, t1, tile_body, 0)

    acc = jnp.zeros((GRID_CHUNK, OUT_CH), jnp.float32)
    for kk in range(27):
        acc = acc + jnp.dot(
            agg_ref[kk * GRID_CHUNK:(kk + 1) * GRID_CHUNK, :].astype(
                jnp.bfloat16),
            wk_ref[kk * IN_CH:(kk + 1) * IN_CH, :],
            preferred_element_type=jnp.float32)
    nbr = jnp.maximum(jnp.sum(nbr_ref[...], axis=1), 1.0)
    out_ref[...] = acc / nbr.reshape(GRID_CHUNK, 1)


def _cconv(starts, feat, pos_t, wk_flat, grid_pos_t, n_chunks):
    grid_spec = pltpu.PrefetchScalarGridSpec(
        num_scalar_prefetch=1,
        grid=(n_chunks,),
        in_specs=[
            pl.BlockSpec((3, GRID_CHUNK), lambda i, s: (0, i)),
            pl.BlockSpec((3, N_PTS), lambda i, s: (0, 0)),
            pl.BlockSpec((N_PTS, IN_CH), lambda i, s: (0, 0)),
            pl.BlockSpec((27 * IN_CH, OUT_CH), lambda i, s: (0, 0)),
        ],
        out_specs=pl.BlockSpec((GRID_CHUNK, OUT_CH), lambda i, s: (i, 0)),
        scratch_shapes=[
            pltpu.VMEM((27 * GRID_CHUNK, IN_CH), jnp.float32),
            pltpu.VMEM((GRID_CHUNK, PT_TILE), jnp.float32),
        ],
    )
    out = pl.pallas_call(
        _cconv_kernel,
        grid_spec=grid_spec,
        out_shape=jax.ShapeDtypeStruct((n_chunks * GRID_CHUNK, OUT_CH),
                                       jnp.float32),
    )(starts, grid_pos_t, pos_t, feat, wk_flat)
    return out


def _full_pipeline(p, feat_in, wk, grid_pos_t, n_chunks):
    rank, starts = _bin_ranks(p[:, 0].reshape(1, N_PTS))
    data = jnp.concatenate(
        [p, jnp.zeros((N_PTS, 16 - 3), jnp.float32), feat_in,
         jnp.zeros((N_PTS, DATA_COLS - 48), jnp.float32)], axis=1)
    sorted_data = _sc_scatter_rows(data, rank)
    pos_t = sorted_data[:, 0:3].T
    feat = sorted_data[:, 16:48].astype(jnp.bfloat16)
    wk_flat = wk.reshape(27 * IN_CH, OUT_CH).astype(jnp.bfloat16)
    return _cconv(starts.reshape(32), feat, pos_t, wk_flat, grid_pos_t,
                  n_chunks)


def kernel(input, pos, kernel, grid_pos):
    p = pos[0]
    grid_pos_t = grid_pos.T

    n_total = NX * NX * NX // GRID_CHUNK
    out = _full_pipeline(p, input[0], kernel, grid_pos_t, n_total)
    grid_feat = out.reshape(1, NX, NX, NX, OUT_CH)
    return jnp.transpose(grid_feat, (0, 4, 1, 2, 3))


# speedup vs baseline: 2.4167x; 1.0111x over previous
import functools

import jax
import jax.numpy as jnp
from jax.experimental import pallas as pl
from jax.experimental.pallas import tpu as pltpu
from jax.experimental.pallas import tpu_sc as plsc

NX = 16
IN_CH = 32
OUT_CH = 64
N_PTS = 4096
DX = 2.0 / NX
RADIUS = DX * 2.5
EPS = 1e-12

GRID_CHUNK = 256
PT_TILE = 256
N_TILES = N_PTS // PT_TILE


def _taps(u):
    w0 = jnp.maximum(0.0, -u)
    w1 = 1.0 - jnp.abs(u)
    w2 = jnp.maximum(0.0, u)
    return w0, w1, w2


_ATAN_C = (0.99999611, -0.33317368, 0.19807815, -0.13233339,
           0.07962363, -0.03360418, 0.00681178)


def _atan_poly(t):
    t2 = t * t
    acc = jnp.float32(_ATAN_C[-1])
    for c in _ATAN_C[-2::-1]:
        acc = jnp.float32(c) + t2 * acc
    return t * acc


def _ball_to_cube(x, y, z, rxy2, sq, zz):
    norm = jnp.sqrt(sq + EPS)
    polar = (1.25 * zz) > rxy2
    s_pol = jnp.sqrt(3.0 * norm / (norm + jnp.abs(z) + EPS))
    cx_p = x * s_pol
    cy_p = y * s_pol
    cz_p = jnp.sign(z) * norm
    rxy = jnp.sqrt(rxy2 + EPS)
    s_eq = norm / rxy
    cx_e = x * s_eq
    cy_e = y * s_eq
    cz_e = 1.5 * z
    cx = jnp.where(polar, cx_p, cx_e)
    cy = jnp.where(polar, cy_p, cy_e)
    cz = jnp.where(polar, cz_p, cz_e)
    degen = sq < 1e-10
    cx = jnp.where(degen, 0.0, cx)
    cy = jnp.where(degen, 0.0, cy)
    cz = jnp.where(degen, 0.0, cz)
    rsq = cx * cx + cy * cy
    rn = jnp.sqrt(rsq + EPS)
    cond = cx * cx >= cy * cy
    safe_cx = jnp.where(jnp.abs(cx) > 1e-6, cx, 1.0)
    safe_cy = jnp.where(jnp.abs(cy) > 1e-6, cy, 1.0)
    pi4 = 4.0 / jnp.pi
    sgn_cx = jnp.sign(cx)
    sgn_cy = jnp.sign(cy)
    u1 = sgn_cx * rn
    v1 = sgn_cx * rn * pi4 * _atan_poly(cy / safe_cx)
    v2 = sgn_cy * rn
    u2 = sgn_cy * rn * pi4 * _atan_poly(cx / safe_cy)
    u = jnp.where(cond, u1, u2)
    v = jnp.where(cond, v1, v2)
    small = rsq < 1e-10
    u = jnp.where(small, 0.0, u)
    v = jnp.where(small, 0.0, v)
    return u, v, cz


SORT_BLK = 512
DATA_COLS = 128
SCAT_WIN = 128


def _bin_kernel(pos_x_ref, rank_ref, starts_ref):
    x = pos_x_ref[0:1, :]
    cf = jnp.clip(jnp.floor((x + 1.0) * (NX / 2.0)), 0.0, NX - 1.0)
    c_col = cf.reshape(N_PTS, 1).astype(jnp.int32)
    lane16 = jax.lax.broadcasted_iota(jnp.int32, (N_PTS, NX), 1)
    onehot = jnp.where(c_col == lane16, 1.0, 0.0)

    r_i = jax.lax.broadcasted_iota(jnp.int32, (SORT_BLK, SORT_BLK), 0)
    c_i = jax.lax.broadcasted_iota(jnp.int32, (SORT_BLK, SORT_BLK), 1)
    tri = jnp.where(r_i >= c_i, 1.0, 0.0).astype(jnp.bfloat16)

    carry = jnp.zeros((1, NX), jnp.float32)
    cums_blocks = []
    for b in range(N_PTS // SORT_BLK):
        blk = onehot[b * SORT_BLK:(b + 1) * SORT_BLK, :]
        within = jnp.dot(tri, blk.astype(jnp.bfloat16),
                         preferred_element_type=jnp.float32)
        cums_blocks.append(within + carry)
        carry = carry + within[SORT_BLK - 1:SORT_BLK, :]
    cums = jnp.concatenate(cums_blocks, axis=0)

    tot_col = carry.reshape(NX, 1)
    t_sub = jax.lax.broadcasted_iota(jnp.int32, (NX, 32), 0)
    s_lane = jax.lax.broadcasted_iota(jnp.int32, (NX, 32), 1)
    starts32 = jnp.sum(jnp.where(t_sub < s_lane, tot_col, 0.0), axis=0,
                       keepdims=True)
    starts_ref[...] = starts32.astype(jnp.int32)

    start_g = jnp.sum(onehot * starts32[:, 0:NX], axis=1, keepdims=True)
    dup_g = jnp.sum(onehot * cums, axis=1, keepdims=True) - 1.0
    rank_col = start_g + dup_g
    rank_ref[...] = rank_col.reshape(1, N_PTS).astype(jnp.int32)


@jax.jit
def _bin_ranks(pos_x):
    return pl.pallas_call(
        _bin_kernel,
        in_specs=[pl.BlockSpec((1, N_PTS), lambda: (0, 0))],
        out_specs=[pl.BlockSpec((1, N_PTS), lambda: (0, 0)),
                   pl.BlockSpec((1, 32), lambda: (0, 0))],
        out_shape=[jax.ShapeDtypeStruct((1, N_PTS), jnp.int32),
                   jax.ShapeDtypeStruct((1, 32), jnp.int32)],
    )(pos_x)


@jax.jit
def _sc_scatter_rows(data, rank):
    mesh = plsc.VectorSubcoreMesh(core_axis_name="core",
                                  subcore_axis_name="subcore")

    @functools.partial(
        pl.kernel,
        out_type=jax.ShapeDtypeStruct((N_PTS, DATA_COLS), jnp.float32),
        mesh=mesh)
    def scatter_kernel(x_hbm, i_hbm, o_hbm):
        def body(x_vmem, i_vmem):
            pltpu.sync_copy(x_vmem, o_hbm.at[i_vmem.at[0]])

        pltpu.emit_pipeline(
            body,
            grid=(N_PTS // SCAT_WIN,),
            in_specs=[pl.BlockSpec((SCAT_WIN, DATA_COLS),
                                   index_map=lambda i: (i, 0)),
                      pl.BlockSpec((1, SCAT_WIN),
                                   index_map=lambda i: (0, i))],
            out_specs=[],
            core_axis_name=("core", "subcore"),
            dimension_semantics=(pltpu.PARALLEL,),
        )(x_hbm, i_hbm)

    return scatter_kernel(data, rank)


def _cconv_kernel(starts_ref, gp_ref, pos_ref, feat_ref, wk_ref, out_ref,
                  agg_ref, nbr_ref, w_ref):
    agg_ref[...] = jnp.zeros_like(agg_ref)
    nbr_ref[...] = jnp.zeros_like(nbr_ref)

    gx = gp_ref[0, :].reshape(GRID_CHUNK, 1)
    gy = gp_ref[1, :].reshape(GRID_CHUNK, 1)
    gz = gp_ref[2, :].reshape(GRID_CHUNK, 1)
    inv_r = 1.0 / RADIUS

    ix = ((gp_ref[0, 0] + 1.0) * (NX / 2.0)).astype(jnp.int32)
    lo = starts_ref[jnp.maximum(ix - 2, 0)]
    hi = starts_ref[jnp.minimum(ix + 3, NX)]
    t0 = lo // PT_TILE
    t1 = (hi + PT_TILE - 1) // PT_TILE

    def tile_body(t, _):
        sl = pl.ds(t * PT_TILE, PT_TILE)
        px = pos_ref[0, sl].reshape(1, PT_TILE)
        py = pos_ref[1, sl].reshape(1, PT_TILE)
        pz = pos_ref[2, sl].reshape(1, PT_TILE)
        feat_t = feat_ref[sl, :]

        x = (px - gx) * inv_r
        y = (py - gy) * inv_r
        z = (pz - gz) * inv_r
        rxy2 = x * x + y * y
        zz = z * z
        sq = rxy2 + zz
        maskf = jnp.where(sq <= 1.0, 1.0, 0.0)
        nbr_ref[...] += maskf

        u, v, w = _ball_to_cube(x, y, z, rxy2, sq, zz)
        wx = _taps(u)
        wy = _taps(v)
        wz = _taps(w)
        wzm = [(wz[k] * maskf).astype(jnp.bfloat16) for k in range(3)]
        for i in range(3):
            for j in range(3):
                wxy = (wx[i] * wy[j]).astype(jnp.bfloat16)
                for k in range(3):
                    kk = (i * 3 + j) * 3 + k
                    w_ref[kk * GRID_CHUNK:(kk + 1) * GRID_CHUNK, :] = (
                        wxy * wzm[k])
        agg_ref[...] += jnp.dot(w_ref[...], feat_t,
                                preferred_element_type=jnp.float32)
        return _

    jax.lax.fori_loop(t0, t1, tile_body, 0)

    acc = jnp.zeros((GRID_CHUNK, OUT_CH), jnp.float32)
    for kk in range(27):
        acc = acc + jnp.dot(
            agg_ref[kk * GRID_CHUNK:(kk + 1) * GRID_CHUNK, :].astype(
                jnp.bfloat16),
            wk_ref[kk * IN_CH:(kk + 1) * IN_CH, :],
            preferred_element_type=jnp.float32)
    nbr = jnp.maximum(jnp.sum(nbr_ref[...], axis=1), 1.0)
    out_ref[...] = acc / nbr.reshape(GRID_CHUNK, 1)


def _cconv(starts, feat, pos_t, wk_flat, grid_pos_t, n_chunks):
    grid_spec = pltpu.PrefetchScalarGridSpec(
        num_scalar_prefetch=1,
        grid=(n_chunks,),
        in_specs=[
            pl.BlockSpec((3, GRID_CHUNK), lambda i, s: (0, i)),
            pl.BlockSpec((3, N_PTS), lambda i, s: (0, 0)),
            pl.BlockSpec((N_PTS, IN_CH), lambda i, s: (0, 0)),
            pl.BlockSpec((27 * IN_CH, OUT_CH), lambda i, s: (0, 0)),
        ],
        out_specs=pl.BlockSpec((GRID_CHUNK, OUT_CH), lambda i, s: (i, 0)),
        scratch_shapes=[
            pltpu.VMEM((27 * GRID_CHUNK, IN_CH), jnp.float32),
            pltpu.VMEM((GRID_CHUNK, PT_TILE), jnp.float32),
            pltpu.VMEM((27 * GRID_CHUNK, PT_TILE), jnp.bfloat16),
        ],
    )
    out = pl.pallas_call(
        _cconv_kernel,
        grid_spec=grid_spec,
        out_shape=jax.ShapeDtypeStruct((n_chunks * GRID_CHUNK, OUT_CH),
                                       jnp.float32),
    )(starts, grid_pos_t, pos_t, feat, wk_flat)
    return out


def _full_pipeline(p, feat_in, wk, grid_pos_t, n_chunks):
    rank, starts = _bin_ranks(p[:, 0].reshape(1, N_PTS))
    data = jnp.concatenate(
        [p, jnp.zeros((N_PTS, 16 - 3), jnp.float32), feat_in,
         jnp.zeros((N_PTS, DATA_COLS - 48), jnp.float32)], axis=1)
    sorted_data = _sc_scatter_rows(data, rank)
    pos_t = sorted_data[:, 0:3].T
    feat = sorted_data[:, 16:48].astype(jnp.bfloat16)
    wk_flat = wk.reshape(27 * IN_CH, OUT_CH).astype(jnp.bfloat16)
    return _cconv(starts.reshape(32), feat, pos_t, wk_flat, grid_pos_t,
                  n_chunks)


def kernel(input, pos, kernel, grid_pos):
    p = pos[0]
    grid_pos_t = grid_pos.T

    n_total = NX * NX * NX // GRID_CHUNK
    out = _full_pipeline(p, input[0], kernel, grid_pos_t, n_total)
    grid_feat = out.reshape(1, NX, NX, NX, OUT_CH)
    return jnp.transpose(grid_feat, (0, 4, 1, 2, 3))


# speedup vs baseline: 2.4432x; 1.0110x over previous
import functools

import jax
import jax.numpy as jnp
from jax.experimental import pallas as pl
from jax.experimental.pallas import tpu as pltpu
from jax.experimental.pallas import tpu_sc as plsc

NX = 16
IN_CH = 32
OUT_CH = 64
N_PTS = 4096
DX = 2.0 / NX
RADIUS = DX * 2.5
EPS = 1e-12

GRID_CHUNK = 256
PT_TILE = 256
N_TILES = N_PTS // PT_TILE


def _taps(u):
    w0 = jnp.maximum(0.0, -u)
    w1 = 1.0 - jnp.abs(u)
    w2 = jnp.maximum(0.0, u)
    return w0, w1, w2


_ATAN_C = (0.99999611, -0.33317368, 0.19807815, -0.13233339,
           0.07962363, -0.03360418, 0.00681178)


def _atan_poly(t):
    t2 = t * t
    acc = jnp.float32(_ATAN_C[-1])
    for c in _ATAN_C[-2::-1]:
        acc = jnp.float32(c) + t2 * acc
    return t * acc


def _ball_to_cube(x, y, z, rxy2, sq, zz):
    norm = jnp.sqrt(sq + EPS)
    polar = (1.25 * zz) > rxy2
    s_pol = jnp.sqrt(3.0 * norm / (norm + jnp.abs(z) + EPS))
    cx_p = x * s_pol
    cy_p = y * s_pol
    cz_p = jnp.sign(z) * norm
    s_eq = norm * jax.lax.rsqrt(rxy2 + EPS)
    cx_e = x * s_eq
    cy_e = y * s_eq
    cz_e = 1.5 * z
    cx = jnp.where(polar, cx_p, cx_e)
    cy = jnp.where(polar, cy_p, cy_e)
    cz = jnp.where(polar, cz_p, cz_e)
    degen = sq < 1e-10
    cx = jnp.where(degen, 0.0, cx)
    cy = jnp.where(degen, 0.0, cy)
    cz = jnp.where(degen, 0.0, cz)
    rsq = cx * cx + cy * cy
    rn = jnp.sqrt(rsq + EPS)
    cond = cx * cx >= cy * cy
    safe_cx = jnp.where(jnp.abs(cx) > 1e-6, cx, 1.0)
    safe_cy = jnp.where(jnp.abs(cy) > 1e-6, cy, 1.0)
    pi4 = 4.0 / jnp.pi
    sgn_cx = jnp.sign(cx)
    sgn_cy = jnp.sign(cy)
    u1 = sgn_cx * rn
    v1 = sgn_cx * rn * pi4 * _atan_poly(cy / safe_cx)
    v2 = sgn_cy * rn
    u2 = sgn_cy * rn * pi4 * _atan_poly(cx / safe_cy)
    u = jnp.where(cond, u1, u2)
    v = jnp.where(cond, v1, v2)
    small = rsq < 1e-10
    u = jnp.where(small, 0.0, u)
    v = jnp.where(small, 0.0, v)
    return u, v, cz


SORT_BLK = 512
DATA_COLS = 128
SCAT_WIN = 128


def _bin_kernel(pos_x_ref, rank_ref, starts_ref):
    x = pos_x_ref[0:1, :]
    cf = jnp.clip(jnp.floor((x + 1.0) * (NX / 2.0)), 0.0, NX - 1.0)
    c_col = cf.reshape(N_PTS, 1).astype(jnp.int32)
    lane16 = jax.lax.broadcasted_iota(jnp.int32, (N_PTS, NX), 1)
    onehot = jnp.where(c_col == lane16, 1.0, 0.0)

    r_i = jax.lax.broadcasted_iota(jnp.int32, (SORT_BLK, SORT_BLK), 0)
    c_i = jax.lax.broadcasted_iota(jnp.int32, (SORT_BLK, SORT_BLK), 1)
    tri = jnp.where(r_i >= c_i, 1.0, 0.0).astype(jnp.bfloat16)

    carry = jnp.zeros((1, NX), jnp.float32)
    cums_blocks = []
    for b in range(N_PTS // SORT_BLK):
        blk = onehot[b * SORT_BLK:(b + 1) * SORT_BLK, :]
        within = jnp.dot(tri, blk.astype(jnp.bfloat16),
                         preferred_element_type=jnp.float32)
        cums_blocks.append(within + carry)
        carry = carry + within[SORT_BLK - 1:SORT_BLK, :]
    cums = jnp.concatenate(cums_blocks, axis=0)

    tot_col = carry.reshape(NX, 1)
    t_sub = jax.lax.broadcasted_iota(jnp.int32, (NX, 32), 0)
    s_lane = jax.lax.broadcasted_iota(jnp.int32, (NX, 32), 1)
    starts32 = jnp.sum(jnp.where(t_sub < s_lane, tot_col, 0.0), axis=0,
                       keepdims=True)
    starts_ref[...] = starts32.astype(jnp.int32)

    start_g = jnp.sum(onehot * starts32[:, 0:NX], axis=1, keepdims=True)
    dup_g = jnp.sum(onehot * cums, axis=1, keepdims=True) - 1.0
    rank_col = start_g + dup_g
    rank_ref[...] = rank_col.reshape(1, N_PTS).astype(jnp.int32)


@jax.jit
def _bin_ranks(pos_x):
    return pl.pallas_call(
        _bin_kernel,
        in_specs=[pl.BlockSpec((1, N_PTS), lambda: (0, 0))],
        out_specs=[pl.BlockSpec((1, N_PTS), lambda: (0, 0)),
                   pl.BlockSpec((1, 32), lambda: (0, 0))],
        out_shape=[jax.ShapeDtypeStruct((1, N_PTS), jnp.int32),
                   jax.ShapeDtypeStruct((1, 32), jnp.int32)],
    )(pos_x)


@jax.jit
def _sc_scatter_rows(data, rank):
    mesh = plsc.VectorSubcoreMesh(core_axis_name="core",
                                  subcore_axis_name="subcore")

    @functools.partial(
        pl.kernel,
        out_type=jax.ShapeDtypeStruct((N_PTS, DATA_COLS), jnp.float32),
        mesh=mesh)
    def scatter_kernel(x_hbm, i_hbm, o_hbm):
        def body(x_vmem, i_vmem):
            pltpu.sync_copy(x_vmem, o_hbm.at[i_vmem.at[0]])

        pltpu.emit_pipeline(
            body,
            grid=(N_PTS // SCAT_WIN,),
            in_specs=[pl.BlockSpec((SCAT_WIN, DATA_COLS),
                                   index_map=lambda i: (i, 0)),
                      pl.BlockSpec((1, SCAT_WIN),
                                   index_map=lambda i: (0, i))],
            out_specs=[],
            core_axis_name=("core", "subcore"),
            dimension_semantics=(pltpu.PARALLEL,),
        )(x_hbm, i_hbm)

    return scatter_kernel(data, rank)


def _cconv_kernel(starts_ref, gp_ref, pos_ref, feat_ref, wk_ref, out_ref,
                  agg_ref, nbr_ref, w_ref):
    agg_ref[...] = jnp.zeros_like(agg_ref)
    nbr_ref[...] = jnp.zeros_like(nbr_ref)

    gx = gp_ref[0, :].reshape(GRID_CHUNK, 1)
    gy = gp_ref[1, :].reshape(GRID_CHUNK, 1)
    gz = gp_ref[2, :].reshape(GRID_CHUNK, 1)
    inv_r = 1.0 / RADIUS

    ix = ((gp_ref[0, 0] + 1.0) * (NX / 2.0)).astype(jnp.int32)
    lo = starts_ref[jnp.maximum(ix - 2, 0)]
    hi = starts_ref[jnp.minimum(ix + 3, NX)]
    t0 = lo // PT_TILE
    t1 = (hi + PT_TILE - 1) // PT_TILE

    def tile_body(t, _):
        sl = pl.ds(t * PT_TILE, PT_TILE)
        px = pos_ref[0, sl].reshape(1, PT_TILE)
        py = pos_ref[1, sl].reshape(1, PT_TILE)
        pz = pos_ref[2, sl].reshape(1, PT_TILE)
        feat_t = feat_ref[sl, :]

        x = (px - gx) * inv_r
        y = (py - gy) * inv_r
        z = (pz - gz) * inv_r
        rxy2 = x * x + y * y
        zz = z * z
        sq = rxy2 + zz
        maskf = jnp.where(sq <= 1.0, 1.0, 0.0)
        nbr_ref[...] += maskf

        u, v, w = _ball_to_cube(x, y, z, rxy2, sq, zz)
        wx = _taps(u)
        wy = _taps(v)
        wz = _taps(w)
        wzm = [(wz[k] * maskf).astype(jnp.bfloat16) for k in range(3)]
        for i in range(3):
            for j in range(3):
                wxy = (wx[i] * wy[j]).astype(jnp.bfloat16)
                for k in range(3):
                    kk = (i * 3 + j) * 3 + k
                    w_ref[kk * GRID_CHUNK:(kk + 1) * GRID_CHUNK, :] = (
                        wxy * wzm[k])
        agg_ref[...] += jnp.dot(w_ref[...], feat_t,
                                preferred_element_type=jnp.float32)
        return _

    jax.lax.fori_loop(t0, t1, tile_body, 0)

    acc = jnp.zeros((GRID_CHUNK, OUT_CH), jnp.float32)
    for kk in range(27):
        acc = acc + jnp.dot(
            agg_ref[kk * GRID_CHUNK:(kk + 1) * GRID_CHUNK, :].astype(
                jnp.bfloat16),
            wk_ref[kk * IN_CH:(kk + 1) * IN_CH, :],
            preferred_element_type=jnp.float32)
    nbr = jnp.maximum(jnp.sum(nbr_ref[...], axis=1), 1.0)
    out_ref[...] = acc / nbr.reshape(GRID_CHUNK, 1)


def _cconv(starts, feat, pos_t, wk_flat, grid_pos_t, n_chunks):
    grid_spec = pltpu.PrefetchScalarGridSpec(
        num_scalar_prefetch=1,
        grid=(n_chunks,),
        in_specs=[
            pl.BlockSpec((3, GRID_CHUNK), lambda i, s: (0, i)),
            pl.BlockSpec((3, N_PTS), lambda i, s: (0, 0)),
            pl.BlockSpec((N_PTS, IN_CH), lambda i, s: (0, 0)),
            pl.BlockSpec((27 * IN_CH, OUT_CH), lambda i, s: (0, 0)),
        ],
        out_specs=pl.BlockSpec((GRID_CHUNK, OUT_CH), lambda i, s: (i, 0)),
        scratch_shapes=[
            pltpu.VMEM((27 * GRID_CHUNK, IN_CH), jnp.float32),
            pltpu.VMEM((GRID_CHUNK, PT_TILE), jnp.float32),
            pltpu.VMEM((27 * GRID_CHUNK, PT_TILE), jnp.bfloat16),
        ],
    )
    out = pl.pallas_call(
        _cconv_kernel,
        grid_spec=grid_spec,
        out_shape=jax.ShapeDtypeStruct((n_chunks * GRID_CHUNK, OUT_CH),
                                       jnp.float32),
    )(starts, grid_pos_t, pos_t, feat, wk_flat)
    return out


def _full_pipeline(p, feat_in, wk, grid_pos_t, n_chunks):
    rank, starts = _bin_ranks(p[:, 0].reshape(1, N_PTS))
    data = jnp.concatenate(
        [p, jnp.zeros((N_PTS, 16 - 3), jnp.float32), feat_in,
         jnp.zeros((N_PTS, DATA_COLS - 48), jnp.float32)], axis=1)
    sorted_data = _sc_scatter_rows(data, rank)
    pos_t = sorted_data[:, 0:3].T
    feat = sorted_data[:, 16:48].astype(jnp.bfloat16)
    wk_flat = wk.reshape(27 * IN_CH, OUT_CH).astype(jnp.bfloat16)
    return _cconv(starts.reshape(32), feat, pos_t, wk_flat, grid_pos_t,
                  n_chunks)


def kernel(input, pos, kernel, grid_pos):
    p = pos[0]
    grid_pos_t = grid_pos.T

    n_total = NX * NX * NX // GRID_CHUNK
    out = _full_pipeline(p, input[0], kernel, grid_pos_t, n_total)
    grid_feat = out.reshape(1, NX, NX, NX, OUT_CH)
    return jnp.transpose(grid_feat, (0, 4, 1, 2, 3))


# speedup vs baseline: 2.4464x; 1.0013x over previous
import functools

import jax
import jax.numpy as jnp
from jax.experimental import pallas as pl
from jax.experimental.pallas import tpu as pltpu
from jax.experimental.pallas import tpu_sc as plsc

NX = 16
IN_CH = 32
OUT_CH = 64
N_PTS = 4096
DX = 2.0 / NX
RADIUS = DX * 2.5
EPS = 1e-12

GRID_CHUNK = 256
PT_TILE = 256


def _taps(u):
    w0 = jnp.maximum(0.0, -u)
    w1 = 1.0 - jnp.abs(u)
    w2 = jnp.maximum(0.0, u)
    return w0, w1, w2


_ATAN_C = (0.99999611, -0.33317368, 0.19807815, -0.13233339,
           0.07962363, -0.03360418, 0.00681178)


def _atan_poly(t):
    t2 = t * t
    acc = jnp.float32(_ATAN_C[-1])
    for c in _ATAN_C[-2::-1]:
        acc = jnp.float32(c) + t2 * acc
    return t * acc


def _ball_to_cube(x, y, z, rxy2, sq, zz):
    norm = jnp.sqrt(sq + EPS)
    polar = (1.25 * zz) > rxy2
    s_pol = jnp.sqrt(3.0 * norm / (norm + jnp.abs(z) + EPS))
    cx_p = x * s_pol
    cy_p = y * s_pol
    cz_p = jnp.sign(z) * norm
    s_eq = norm * jax.lax.rsqrt(rxy2 + EPS)
    cx_e = x * s_eq
    cy_e = y * s_eq
    cz_e = 1.5 * z
    cx = jnp.where(polar, cx_p, cx_e)
    cy = jnp.where(polar, cy_p, cy_e)
    cz = jnp.where(polar, cz_p, cz_e)
    degen = sq < 1e-10
    cx = jnp.where(degen, 0.0, cx)
    cy = jnp.where(degen, 0.0, cy)
    cz = jnp.where(degen, 0.0, cz)
    rsq = cx * cx + cy * cy
    rn = jnp.sqrt(rsq + EPS)
    cond = cx * cx >= cy * cy
    safe_cx = jnp.where(jnp.abs(cx) > 1e-6, cx, 1.0)
    safe_cy = jnp.where(jnp.abs(cy) > 1e-6, cy, 1.0)
    pi4 = 4.0 / jnp.pi
    sgn_cx = jnp.sign(cx)
    sgn_cy = jnp.sign(cy)
    u1 = sgn_cx * rn
    v1 = sgn_cx * rn * pi4 * _atan_poly(cy / safe_cx)
    v2 = sgn_cy * rn
    u2 = sgn_cy * rn * pi4 * _atan_poly(cx / safe_cy)
    u = jnp.where(cond, u1, u2)
    v = jnp.where(cond, v1, v2)
    small = rsq < 1e-10
    u = jnp.where(small, 0.0, u)
    v = jnp.where(small, 0.0, v)
    return u, v, cz


SORT_BLK = 512
DATA_COLS = 128
SCAT_WIN = 128


def _bin_kernel(pos_x_ref, rank_ref, starts_ref):
    x = pos_x_ref[0:1, :]
    cf = jnp.clip(jnp.floor((x + 1.0) * (NX / 2.0)), 0.0, NX - 1.0)
    c_col = cf.reshape(N_PTS, 1).astype(jnp.int32)
    lane16 = jax.lax.broadcasted_iota(jnp.int32, (N_PTS, NX), 1)
    onehot = jnp.where(c_col == lane16, 1.0, 0.0)

    r_i = jax.lax.broadcasted_iota(jnp.int32, (SORT_BLK, SORT_BLK), 0)
    c_i = jax.lax.broadcasted_iota(jnp.int32, (SORT_BLK, SORT_BLK), 1)
    tri = jnp.where(r_i >= c_i, 1.0, 0.0).astype(jnp.bfloat16)

    carry = jnp.zeros((1, NX), jnp.float32)
    cums_blocks = []
    for b in range(N_PTS // SORT_BLK):
        blk = onehot[b * SORT_BLK:(b + 1) * SORT_BLK, :]
        within = jnp.dot(tri, blk.astype(jnp.bfloat16),
                         preferred_element_type=jnp.float32)
        cums_blocks.append(within + carry)
        carry = carry + within[SORT_BLK - 1:SORT_BLK, :]
    cums = jnp.concatenate(cums_blocks, axis=0)

    tot_col = carry.reshape(NX, 1)
    t_sub = jax.lax.broadcasted_iota(jnp.int32, (NX, 32), 0)
    s_lane = jax.lax.broadcasted_iota(jnp.int32, (NX, 32), 1)
    starts32 = jnp.sum(jnp.where(t_sub < s_lane, tot_col, 0.0), axis=0,
                       keepdims=True)
    starts_ref[...] = starts32.astype(jnp.int32)

    start_g = jnp.sum(onehot * starts32[:, 0:NX], axis=1, keepdims=True)
    dup_g = jnp.sum(onehot * cums, axis=1, keepdims=True) - 1.0
    rank_col = start_g + dup_g
    rank_ref[...] = rank_col.reshape(1, N_PTS).astype(jnp.int32)


@jax.jit
def _bin_ranks(pos_x):
    return pl.pallas_call(
        _bin_kernel,
        in_specs=[pl.BlockSpec((1, N_PTS), lambda: (0, 0))],
        out_specs=[pl.BlockSpec((1, N_PTS), lambda: (0, 0)),
                   pl.BlockSpec((1, 32), lambda: (0, 0))],
        out_shape=[jax.ShapeDtypeStruct((1, N_PTS), jnp.int32),
                   jax.ShapeDtypeStruct((1, 32), jnp.int32)],
    )(pos_x)


@jax.jit
def _sc_scatter_rows(data, rank):
    mesh = plsc.VectorSubcoreMesh(core_axis_name="core",
                                  subcore_axis_name="subcore")

    @functools.partial(
        pl.kernel,
        out_type=jax.ShapeDtypeStruct((N_PTS, DATA_COLS), jnp.float32),
        mesh=mesh)
    def scatter_kernel(x_hbm, i_hbm, o_hbm):
        def body(x_vmem, i_vmem):
            pltpu.sync_copy(x_vmem, o_hbm.at[i_vmem.at[0]])

        pltpu.emit_pipeline(
            body,
            grid=(N_PTS // SCAT_WIN,),
            in_specs=[pl.BlockSpec((SCAT_WIN, DATA_COLS),
                                   index_map=lambda i: (i, 0)),
                      pl.BlockSpec((1, SCAT_WIN),
                                   index_map=lambda i: (0, i))],
            out_specs=[],
            core_axis_name=("core", "subcore"),
            dimension_semantics=(pltpu.PARALLEL,),
        )(x_hbm, i_hbm)

    return scatter_kernel(data, rank)


def _cconv_kernel(starts_ref, gp_ref, pos_ref, feat_ref, wk_ref, out_ref,
                  agg_ref, nbr_ref, w_ref):
    agg_ref[...] = jnp.zeros_like(agg_ref)
    nbr_ref[...] = jnp.zeros_like(nbr_ref)

    gx = gp_ref[0, :].reshape(GRID_CHUNK, 1)
    gy = gp_ref[1, :].reshape(GRID_CHUNK, 1)
    gz = gp_ref[2, :].reshape(GRID_CHUNK, 1)
    inv_r = 1.0 / RADIUS

    ix = ((gp_ref[0, 0] + 1.0) * (NX / 2.0)).astype(jnp.int32)
    lo = starts_ref[jnp.maximum(ix - 2, 0)]
    hi = starts_ref[jnp.minimum(ix + 3, NX)]
    t0 = lo // PT_TILE
    t1 = (hi + PT_TILE - 1) // PT_TILE

    def tile_body(t, _):
        sl = pl.ds(t * PT_TILE, PT_TILE)
        px = pos_ref[0, sl].reshape(1, PT_TILE)
        py = pos_ref[1, sl].reshape(1, PT_TILE)
        pz = pos_ref[2, sl].reshape(1, PT_TILE)
        feat_t = feat_ref[sl, :]

        x = (px - gx) * inv_r
        y = (py - gy) * inv_r
        z = (pz - gz) * inv_r
        rxy2 = x * x + y * y
        zz = z * z
        sq = rxy2 + zz
        maskf = jnp.where(sq <= 1.0, 1.0, 0.0)
        nbr_ref[...] += maskf

        u, v, w = _ball_to_cube(x, y, z, rxy2, sq, zz)
        wx = _taps(u)
        wy = _taps(v)
        wz = _taps(w)
        wzm = [(wz[k] * maskf).astype(jnp.bfloat16) for k in range(3)]
        for i in range(3):
            for j in range(3):
                wxy = (wx[i] * wy[j]).astype(jnp.bfloat16)
                for k in range(3):
                    kk = (i * 3 + j) * 3 + k
                    w_ref[kk * GRID_CHUNK:(kk + 1) * GRID_CHUNK, :] = (
                        wxy * wzm[k])
        agg_ref[...] += jnp.dot(w_ref[...], feat_t,
                                preferred_element_type=jnp.float32)
        return _

    jax.lax.fori_loop(t0, t1, tile_body, 0)

    acc = jnp.zeros((GRID_CHUNK, OUT_CH), jnp.float32)
    for kk in range(27):
        acc = acc + jnp.dot(
            agg_ref[kk * GRID_CHUNK:(kk + 1) * GRID_CHUNK, :].astype(
                jnp.bfloat16),
            wk_ref[kk * IN_CH:(kk + 1) * IN_CH, :],
            preferred_element_type=jnp.float32)
    nbr = jnp.maximum(jnp.sum(nbr_ref[...], axis=1), 1.0)
    out_ref[...] = acc / nbr.reshape(GRID_CHUNK, 1)


def _cconv(starts, feat, pos_t, wk_flat, grid_pos_t, n_chunks):
    grid_spec = pltpu.PrefetchScalarGridSpec(
        num_scalar_prefetch=1,
        grid=(n_chunks,),
        in_specs=[
            pl.BlockSpec((3, GRID_CHUNK), lambda i, s: (0, i)),
            pl.BlockSpec((3, N_PTS), lambda i, s: (0, 0)),
            pl.BlockSpec((N_PTS, IN_CH), lambda i, s: (0, 0)),
            pl.BlockSpec((27 * IN_CH, OUT_CH), lambda i, s: (0, 0)),
        ],
        out_specs=pl.BlockSpec((GRID_CHUNK, OUT_CH), lambda i, s: (i, 0)),
        scratch_shapes=[
            pltpu.VMEM((27 * GRID_CHUNK, IN_CH), jnp.float32),
            pltpu.VMEM((GRID_CHUNK, PT_TILE), jnp.float32),
            pltpu.VMEM((27 * GRID_CHUNK, PT_TILE), jnp.bfloat16),
        ],
    )
    out = pl.pallas_call(
        _cconv_kernel,
        grid_spec=grid_spec,
        out_shape=jax.ShapeDtypeStruct((n_chunks * GRID_CHUNK, OUT_CH),
                                       jnp.float32),
    )(starts, grid_pos_t, pos_t, feat, wk_flat)
    return out


def _full_pipeline(p, feat_in, wk, grid_pos_t, n_chunks):
    rank, starts = _bin_ranks(p[:, 0].reshape(1, N_PTS))
    data = jnp.concatenate(
        [p, jnp.zeros((N_PTS, 16 - 3), jnp.float32), feat_in,
         jnp.zeros((N_PTS, DATA_COLS - 48), jnp.float32)], axis=1)
    sorted_data = _sc_scatter_rows(data, rank)
    pos_t = sorted_data[:, 0:3].T
    feat = sorted_data[:, 16:48].astype(jnp.bfloat16)
    wk_flat = wk.reshape(27 * IN_CH, OUT_CH).astype(jnp.bfloat16)
    return _cconv(starts.reshape(32), feat, pos_t, wk_flat, grid_pos_t,
                  n_chunks)


def kernel(input, pos, kernel, grid_pos):
    p = pos[0]
    grid_pos_t = grid_pos.T

    n_total = NX * NX * NX // GRID_CHUNK
    out = _full_pipeline(p, input[0], kernel, grid_pos_t, n_total)
    grid_feat = out.reshape(1, NX, NX, NX, OUT_CH)
    return jnp.transpose(grid_feat, (0, 4, 1, 2, 3))
